# Initial kernel scaffold; baseline (speedup 1.0000x reference)
import jax
import jax.numpy as jnp
from jax.experimental import pallas as pl


def kernel(input_ids, proj_w, ln_gamma, ln_beta):
    raise NotImplementedError("write your pallas kernel here")


# speedup vs baseline: 3.8931x; 3.8931x over previous
import jax
import jax.numpy as jnp
from jax import lax
from jax.experimental import pallas as pl


def _proj_ln_body(ids_ref, wt_ref, g_ref, b_ref, o_ref):
    ids = ids_ref[...]
    tb, w = ids.shape
    n = wt_ref.shape[0]
    iota = lax.broadcasted_iota(jnp.int32, (tb, n), 1)
    acc = ids[:, 0][:, None] == iota
    for j in range(1, w):
        acc = acc | (ids[:, j][:, None] == iota)
    onehot = acc.astype(wt_ref.dtype)
    x = jnp.dot(onehot, wt_ref[...], preferred_element_type=jnp.float32)
    mean = jnp.mean(x, axis=1, keepdims=True)
    var = jnp.mean(x * x, axis=1, keepdims=True) - mean * mean
    inv = lax.rsqrt(var + 1e-5)
    o_ref[...] = (x - mean) * inv * g_ref[...] + b_ref[...]


def kernel(input_ids, proj_w, ln_gamma, ln_beta):
    b, s, w = input_ids.shape
    h, n = proj_w.shape
    t = b * s
    ids = input_ids.reshape(t, w).astype(jnp.int32)
    wt = proj_w.T.astype(jnp.bfloat16)
    tb = 512
    while t % tb:
        tb //= 2
    grid = t // tb
    out = pl.pallas_call(
        _proj_ln_body,
        grid=(grid,),
        in_specs=[
            pl.BlockSpec((tb, w), lambda i: (i, 0)),
            pl.BlockSpec((n, h), lambda i: (0, 0)),
            pl.BlockSpec((1, h), lambda i: (0, 0)),
            pl.BlockSpec((1, h), lambda i: (0, 0)),
        ],
        out_specs=pl.BlockSpec((tb, h), lambda i: (i, 0)),
        out_shape=jax.ShapeDtypeStruct((t, h), jnp.float32),
    )(ids, wt, ln_gamma.reshape(1, h), ln_beta.reshape(1, h))
    return out.reshape(b, s, h)


# speedup vs baseline: 8.7811x; 2.2555x over previous
import functools

import jax
import jax.numpy as jnp
from jax import lax
from jax.experimental import pallas as pl
from jax.experimental.pallas import tpu as pltpu
from jax.experimental.pallas import tpu_sc as plsc

_NC, _NS = 2, 16
_NW = _NC * _NS
_GB = 16
_KC = 3
_NBUF = 2


def _sc_scatter_body(ids_hbm, zeros_hbm, out_hbm, ids_v, buf0, buf1, sem0, sem1):
    wid = lax.axis_index("s") * _NC + lax.axis_index("c")
    tpw = ids_v.shape[0] // (_KC * 16)
    n = zeros_hbm.shape[0] // _GB
    base = wid * tpw
    ng = tpw // _GB
    pltpu.sync_copy(ids_hbm.at[pl.ds(base * (_KC * 16), tpw * (_KC * 16))], ids_v)
    pltpu.sync_copy(zeros_hbm, buf0)
    pltpu.sync_copy(zeros_hbm, buf1)
    zeros16 = jnp.zeros((16,), jnp.float32)
    ones16 = jnp.ones((16,), jnp.float32)
    bufs = (buf0, buf1)
    sems = (sem0, sem1)

    def _scatter_group(buf, g, val):
        for r in range(_GB):
            lt = g * _GB + r
            for k in range(_KC):
                idx = ids_v[pl.ds(lt * (_KC * 16) + k * 16, 16)] + (r * n)
                plsc.store_scatter(buf, [idx], val)

    def body(i, carry):
        for b in range(_NBUF):
            buf, sem = bufs[b], sems[b]
            g = i * _NBUF + b

            @pl.when(i > 0)
            def _():
                pltpu.make_async_copy(
                    buf, out_hbm.at[pl.ds(0, _GB * n)], sem).wait()
                _scatter_group(buf, g - _NBUF, zeros16)

            _scatter_group(buf, g, ones16)
            pltpu.async_copy(
                buf, out_hbm.at[pl.ds((base + g * _GB) * n, _GB * n)], sem)
        return carry

    lax.fori_loop(0, ng // _NBUF, body, 0)
    for b in range(_NBUF):
        pltpu.make_async_copy(
            bufs[b], out_hbm.at[pl.ds(0, _GB * n)], sems[b]).wait()


def _build_onehot_sc(ids_pad, t, n):
    mesh = plsc.VectorSubcoreMesh(
        core_axis_name="c", subcore_axis_name="s",
        num_cores=_NC, num_subcores=_NS)
    tpw = t // _NW
    zeros = jnp.zeros((_GB * n,), jnp.float32)
    sc_kernel = functools.partial(
        pl.kernel,
        out_type=jax.ShapeDtypeStruct((t * n,), jnp.float32),
        mesh=mesh,
        scratch_types=[
            pltpu.VMEM((tpw * _KC * 16,), jnp.int32),
            pltpu.VMEM((_GB * n,), jnp.float32),
            pltpu.VMEM((_GB * n,), jnp.float32),
            pltpu.SemaphoreType.DMA,
            pltpu.SemaphoreType.DMA,
        ],
        compiler_params=pltpu.CompilerParams(needs_layout_passes=False),
    )(_sc_scatter_body)
    return sc_kernel(ids_pad, zeros).reshape(t, n)


def _proj_ln_body(oh_ref, wt_ref, g_ref, b_ref, o_ref):
    oh = oh_ref[...].astype(wt_ref.dtype)
    x = jnp.dot(oh, wt_ref[...], preferred_element_type=jnp.float32)
    mean = jnp.mean(x, axis=1, keepdims=True)
    var = jnp.mean(x * x, axis=1, keepdims=True) - mean * mean
    inv = lax.rsqrt(var + 1e-5)
    o_ref[...] = (x - mean) * inv * g_ref[...] + b_ref[...]


def kernel(input_ids, proj_w, ln_gamma, ln_beta):
    b, s, w = input_ids.shape
    h, n = proj_w.shape
    t = b * s
    ids = input_ids.reshape(t, w).astype(jnp.int32)
    pad = (-w) % 16
    ids_pad = jnp.concatenate(
        [ids, jnp.broadcast_to(ids[:, :1], (t, pad))], axis=1
    ).reshape(t * (w + pad))
    onehot = _build_onehot_sc(ids_pad, t, n)
    wt = proj_w.T.astype(jnp.bfloat16)
    tb = 512
    while t % tb:
        tb //= 2
    grid = t // tb
    out = pl.pallas_call(
        _proj_ln_body,
        grid=(grid,),
        in_specs=[
            pl.BlockSpec((tb, n), lambda i: (i, 0)),
            pl.BlockSpec((n, h), lambda i: (0, 0)),
            pl.BlockSpec((1, h), lambda i: (0, 0)),
            pl.BlockSpec((1, h), lambda i: (0, 0)),
        ],
        out_specs=pl.BlockSpec((tb, h), lambda i: (i, 0)),
        out_shape=jax.ShapeDtypeStruct((t, h), jnp.float32),
    )(onehot, wt, ln_gamma.reshape(1, h), ln_beta.reshape(1, h))
    return out.reshape(b, s, h)
